# Initial kernel scaffold; baseline (speedup 1.0000x reference)
import jax
import jax.numpy as jnp
from jax.experimental import pallas as pl


def kernel(x, edge_index, W0, b0, W1, b1, Wout, bout):
    raise NotImplementedError("write your pallas kernel here")


# speedup vs baseline: 16.2328x; 16.2328x over previous
import functools

import jax
import jax.numpy as jnp
from jax import lax
from jax.experimental import pallas as pl
from jax.experimental.pallas import tpu as pltpu
from jax.experimental.pallas import tpu_sc as plsc

_LANES = 16
_NT = 16
_NC = 2
_BLK = 1024


def _sc_deg(dst, *, E, NP, R, CH, REM):
    mesh = plsc.VectorSubcoreMesh(core_axis_name="c", subcore_axis_name="s")
    scratch = [
        pltpu.VMEM_SHARED((NP,), jnp.float32),
        pltpu.VMEM((128,), jnp.int32),
        pltpu.VMEM((128,), jnp.float32),
        pltpu.VMEM((128,), jnp.float32),
        pltpu.VMEM((R,), jnp.float32),
        pltpu.VMEM((R, 128), jnp.float32),
    ]
    if REM:
        scratch += [pltpu.VMEM((REM,), jnp.int32), pltpu.VMEM((REM,), jnp.float32)]

    @functools.partial(
        pl.kernel,
        out_type=jax.ShapeDtypeStruct((_NC, NP, 128), jnp.float32),
        mesh=mesh,
        scratch_types=scratch,
    )
    def k(dst_hbm, degb_hbm, sdeg, idxb, ones128, z128, degv, bcast, *rest):
        c = lax.axis_index("c")
        s = lax.axis_index("s")
        base = (c * _NT + s) * (E // (_NC * _NT))
        nbase = s * R
        for j in range(128 // _LANES):
            z128[pl.ds(_LANES * j, _LANES)] = jnp.zeros((_LANES,), jnp.float32)
            ones128[pl.ds(_LANES * j, _LANES)] = jnp.ones((_LANES,), jnp.float32)
        for j in range(R // 128):
            pltpu.sync_copy(z128, sdeg.at[pl.ds(nbase + 128 * j, 128)])
        plsc.subcore_barrier()

        def body(i, carry):
            off = base + i * 128
            pltpu.sync_copy(dst_hbm.at[pl.ds(off, 128)], idxb)
            pltpu.sync_copy(ones128, sdeg.at[idxb], add=True)
            return carry

        lax.fori_loop(0, CH, body, 0)
        if REM:
            idxr, onesr = rest
            for j in range(REM // _LANES):
                onesr[pl.ds(_LANES * j, _LANES)] = jnp.ones((_LANES,), jnp.float32)
            off = base + CH * 128
            pltpu.sync_copy(dst_hbm.at[pl.ds(off, REM)], idxr)
            pltpu.sync_copy(onesr, sdeg.at[idxr], add=True)
        plsc.subcore_barrier()
        pltpu.sync_copy(sdeg.at[pl.ds(nbase, R)], degv)

        def bgrp(g, carry):
            v = degv[pl.ds(_LANES * g, _LANES)]
            for l in range(_LANES):
                row = jnp.zeros((_LANES,), jnp.float32) + v[l]
                brow = bcast.at[_LANES * g + l]
                for j in range(128 // _LANES):
                    brow[pl.ds(_LANES * j, _LANES)] = row
            return carry

        lax.fori_loop(0, R // _LANES, bgrp, 0)
        pltpu.sync_copy(bcast, degb_hbm.at[c, pl.ds(nbase, R)])

    return k(dst)


def _sc_agg(gt, src, dst, *, E, NP, R, CH, REM):
    mesh = plsc.VectorSubcoreMesh(core_axis_name="c", subcore_axis_name="s")
    scratch = [
        pltpu.VMEM_SHARED((NP, 128), jnp.float32),
        pltpu.VMEM((128,), jnp.int32),
        pltpu.VMEM((128,), jnp.int32),
        pltpu.VMEM((128, 128), jnp.float32),
        pltpu.VMEM((128, 128), jnp.float32),
        pltpu.SemaphoreType.DMA,
    ]
    if REM:
        scratch += [
            pltpu.VMEM((REM,), jnp.int32),
            pltpu.VMEM((REM,), jnp.int32),
            pltpu.VMEM((REM, 128), jnp.float32),
        ]

    @functools.partial(
        pl.kernel,
        out_type=jax.ShapeDtypeStruct((_NC, NP, 128), jnp.float32),
        mesh=mesh,
        scratch_types=scratch,
    )
    def k(gt_hbm, src_hbm, dst_hbm, accp_hbm, acc, sidx, didx, zblk, rowsg,
          sem, *rest):
        c = lax.axis_index("c")
        s = lax.axis_index("s")
        base = (c * _NT + s) * (E // (_NC * _NT))
        nbase = s * R

        def zrow(r, carry):
            zr = zblk.at[r]
            for j in range(128 // _LANES):
                zr[pl.ds(_LANES * j, _LANES)] = jnp.zeros((_LANES,), jnp.float32)
            return carry

        lax.fori_loop(0, 128, zrow, 0)
        for j in range(R // 128):
            pltpu.sync_copy(zblk, acc.at[pl.ds(nbase + 128 * j, 128)])
        plsc.subcore_barrier()

        def body(i, carry):
            off = base + i * 128
            pltpu.sync_copy(src_hbm.at[pl.ds(off, 128)], sidx)
            pltpu.sync_copy(dst_hbm.at[pl.ds(off, 128)], didx)
            pltpu.async_copy(gt_hbm.at[sidx], rowsg, sem).wait()
            pltpu.sync_copy(rowsg, acc.at[didx], add=True)
            return carry

        lax.fori_loop(0, CH, body, 0)
        if REM:
            sidxr, didxr, rowsr = rest
            off = base + CH * 128
            pltpu.sync_copy(src_hbm.at[pl.ds(off, REM)], sidxr)
            pltpu.sync_copy(dst_hbm.at[pl.ds(off, REM)], didxr)
            pltpu.async_copy(gt_hbm.at[sidxr], rowsr, sem).wait()
            pltpu.sync_copy(rowsr, acc.at[didxr], add=True)
        plsc.subcore_barrier()
        pltpu.sync_copy(acc.at[pl.ds(nbase, R)], accp_hbm.at[c, pl.ds(nbase, R)])

    return k(gt, src, dst)


def _row_specs(np_, d, n):
    return [pl.BlockSpec((_BLK, d), lambda r: (r, 0)) for _ in range(n)]


def _tc_matmul0(xp, W, br):
    NP, D = xp.shape
    H = W.shape[1]

    def body(x_ref, w_ref, b_ref, o_ref):
        o_ref[...] = (
            jnp.dot(x_ref[...], w_ref[...], preferred_element_type=jnp.float32)
            + b_ref[...]
        )

    return pl.pallas_call(
        body,
        grid=(NP // _BLK,),
        in_specs=[
            pl.BlockSpec((_BLK, D), lambda r: (r, 0)),
            pl.BlockSpec((D, H), lambda r: (0, 0)),
            pl.BlockSpec((1, H), lambda r: (0, 0)),
        ],
        out_specs=pl.BlockSpec((_BLK, H), lambda r: (r, 0)),
        out_shape=jax.ShapeDtypeStruct((NP, H), jnp.float32),
    )(xp, W, br)


def _tc_scale(degb, g0):
    _, NP, D = degb.shape

    def body(d_ref, g_ref, dinv_ref, gt_ref):
        d = d_ref[...]
        dinv = lax.rsqrt(d[0] + d[1] + 1.0)
        dinv_ref[...] = dinv
        gt_ref[...] = g_ref[...] * dinv

    return pl.pallas_call(
        body,
        grid=(NP // _BLK,),
        in_specs=[
            pl.BlockSpec((_NC, _BLK, D), lambda r: (0, r, 0)),
            pl.BlockSpec((_BLK, D), lambda r: (r, 0)),
        ],
        out_specs=[
            pl.BlockSpec((_BLK, D), lambda r: (r, 0)),
            pl.BlockSpec((_BLK, D), lambda r: (r, 0)),
        ],
        out_shape=[
            jax.ShapeDtypeStruct((NP, D), jnp.float32),
            jax.ShapeDtypeStruct((NP, D), jnp.float32),
        ],
    )(degb, g0)


def _tc_layer(accp, gt0, dinvb, W, br):
    _, NP, D = accp.shape
    H = W.shape[1]

    def body(a_ref, gt_ref, dv_ref, w_ref, b_ref, h1_ref, gt1_ref):
        a = a_ref[...]
        dv = dv_ref[...]
        h1 = jnp.maximum(dv * (a[0] + a[1] + gt_ref[...]), 0.0)
        h1_ref[...] = h1
        gt1_ref[...] = (
            jnp.dot(h1, w_ref[...], preferred_element_type=jnp.float32)
            + b_ref[...]
        ) * dv

    return pl.pallas_call(
        body,
        grid=(NP // _BLK,),
        in_specs=[
            pl.BlockSpec((_NC, _BLK, D), lambda r: (0, r, 0)),
            pl.BlockSpec((_BLK, D), lambda r: (r, 0)),
            pl.BlockSpec((_BLK, D), lambda r: (r, 0)),
            pl.BlockSpec((D, H), lambda r: (0, 0)),
            pl.BlockSpec((1, H), lambda r: (0, 0)),
        ],
        out_specs=[
            pl.BlockSpec((_BLK, D), lambda r: (r, 0)),
            pl.BlockSpec((_BLK, H), lambda r: (r, 0)),
        ],
        out_shape=[
            jax.ShapeDtypeStruct((NP, D), jnp.float32),
            jax.ShapeDtypeStruct((NP, H), jnp.float32),
        ],
    )(accp, gt0, dinvb, W, br)


def _tc_final(accp, gt1, dinvb, h1, WoutP, boutP):
    _, NP, D = accp.shape

    def body(a_ref, gt_ref, dv_ref, h1_ref, w_ref, b_ref, o_ref):
        a = a_ref[...]
        h2 = jnp.maximum(dv_ref[...] * (a[0] + a[1] + gt_ref[...]), 0.0)
        h1 = h1_ref[...]
        s1 = jnp.sum(h1 * h1, axis=1, keepdims=True)
        s2 = jnp.sum(h2 * h2, axis=1, keepdims=True)
        m = jnp.maximum(s1, s2)
        e1 = jnp.exp(s1 - m)
        e2 = jnp.exp(s2 - m)
        h = (e1 * h1 + e2 * h2) / (e1 + e2)
        o_ref[...] = (
            jnp.dot(h, w_ref[...], preferred_element_type=jnp.float32)
            + b_ref[...]
        )

    return pl.pallas_call(
        body,
        grid=(NP // _BLK,),
        in_specs=[
            pl.BlockSpec((_NC, _BLK, D), lambda r: (0, r, 0)),
            pl.BlockSpec((_BLK, D), lambda r: (r, 0)),
            pl.BlockSpec((_BLK, D), lambda r: (r, 0)),
            pl.BlockSpec((_BLK, D), lambda r: (r, 0)),
            pl.BlockSpec((D, D), lambda r: (0, 0)),
            pl.BlockSpec((1, D), lambda r: (0, 0)),
        ],
        out_specs=pl.BlockSpec((_BLK, D), lambda r: (r, 0)),
        out_shape=jax.ShapeDtypeStruct((NP, D), jnp.float32),
    )(accp, gt1, dinvb, h1, WoutP, boutP)


def kernel(x, edge_index, W0, b0, W1, b1, Wout, bout):
    N, D = x.shape
    E = edge_index.shape[1]
    H = W0.shape[1]
    C = Wout.shape[1]

    R = -(-N // _NT)
    R = -(-R // 128) * 128
    NP = _NT * R
    ET = E // (_NC * _NT)
    CH, REM = ET // 128, ET % 128

    xp = jnp.pad(x, ((0, NP - N), (0, 0)))
    src = edge_index[0].astype(jnp.int32)
    dst = edge_index[1].astype(jnp.int32)
    b0r = b0.reshape(1, H)
    b1r = b1.reshape(1, H)
    WoutP = jnp.pad(Wout, ((0, 0), (0, D - C)))
    boutP = jnp.pad(bout, (0, D - C)).reshape(1, D)

    degb = _sc_deg(dst, E=E, NP=NP, R=R, CH=CH, REM=REM)
    g0 = _tc_matmul0(xp, W0, b0r)
    dinvb, gt0 = _tc_scale(degb, g0)
    accp1 = _sc_agg(gt0, src, dst, E=E, NP=NP, R=R, CH=CH, REM=REM)
    h1, gt1 = _tc_layer(accp1, gt0, dinvb, W1, b1r)
    accp2 = _sc_agg(gt1, src, dst, E=E, NP=NP, R=R, CH=CH, REM=REM)
    outp = _tc_final(accp2, gt1, dinvb, h1, WoutP, boutP)
    return outp[:N, :C]
